# Initial kernel scaffold; baseline (speedup 1.0000x reference)
import jax
import jax.numpy as jnp
from jax.experimental import pallas as pl


def kernel(transpose_xyz, W1, g1, b1, W2, g2, b2, W3, g3, b3, W4, g4, b4, W5, g5, b5, Wm1, bm1, g6, b6, Wm2, bm2, g7, b7, Wm3, bm3, g8, b8):
    raise NotImplementedError("write your pallas kernel here")


# speedup vs baseline: 10.4687x; 10.4687x over previous
import functools

import jax
import jax.numpy as jnp
from jax import lax
from jax.experimental import pallas as pl
from jax.experimental.pallas import tpu as pltpu
from jax.experimental.pallas import tpu_sc as plsc

B = 8
N = 2048
BN = B * N
K_NN = 20
EPS = 1e-5
NEG = -3.0e38
CP = 128

NW = 32
PTS_W = BN // NW
P_CHUNK = 4
N_CHUNK = PTS_W // P_CHUNK


R_TOPK = 256


def _topk_body(x_ref, xt_ref, idx_ref, pd_ref):
    b = pl.program_id(0)
    rows = x_ref[...]
    xt = xt_ref[0]
    g = jnp.dot(rows, xt, preferred_element_type=jnp.float32)
    xx_r = jnp.sum(rows * rows, axis=1, keepdims=True)
    xx_c = jnp.sum(xt * xt, axis=0, keepdims=True)
    pd_ref[...] = 2.0 * g - xx_r - xx_c
    iota = lax.broadcasted_iota(jnp.int32, (R_TOPK, N), 1)
    outs = []
    for _ in range(K_NN):
        pd = pd_ref[...]
        mx = jnp.max(pd, axis=1, keepdims=True)
        cand = jnp.where(pd == mx, iota, N)
        arg = jnp.min(cand, axis=1, keepdims=True)
        outs.append(arg)
        pd_ref[...] = jnp.where(iota == arg, NEG, pd)
    idx_ref[...] = jnp.concatenate(outs, axis=1) + b * N


def _topk(x_pad, xt_pad):
    grid = (B, N // R_TOPK)
    return pl.pallas_call(
        _topk_body,
        grid=grid,
        in_specs=[
            pl.BlockSpec((R_TOPK, CP), lambda b, i: (b * (N // R_TOPK) + i, 0)),
            pl.BlockSpec((1, CP, N), lambda b, i: (b, 0, 0)),
        ],
        out_specs=pl.BlockSpec((R_TOPK, K_NN), lambda b, i: (b * (N // R_TOPK) + i, 0)),
        out_shape=jax.ShapeDtypeStruct((BN, K_NN), jnp.int32),
        scratch_shapes=[pltpu.VMEM((R_TOPK, N), jnp.float32)],
    )(x_pad, xt_pad)


def _make_sc_gather_simple():
    mesh = plsc.VectorSubcoreMesh(core_axis_name="c", subcore_axis_name="s")

    @functools.partial(
        pl.kernel,
        mesh=mesh,
        out_type=jax.ShapeDtypeStruct((BN * K_NN, CP), jnp.float32),
        scratch_types=[
            pltpu.VMEM((PTS_W * K_NN,), jnp.int32),
            pltpu.VMEM((P_CHUNK * K_NN, CP), jnp.float32),
            pltpu.VMEM((P_CHUNK * K_NN, CP), jnp.float32),
            pltpu.SemaphoreType.DMA,
            pltpu.SemaphoreType.DMA,
        ],
    )
    def sc_kernel(table_hbm, idx_hbm, feat_hbm,
                  idx_v, rows0_v, rows1_v, gsem, wsem):
        wid = lax.axis_index("s") * 2 + lax.axis_index("c")
        rbase = wid * (PTS_W * K_NN)
        pltpu.sync_copy(idx_hbm.at[pl.ds(rbase, PTS_W * K_NN)], idx_v)
        nrow = P_CHUNK * K_NN

        pltpu.async_copy(
            table_hbm.at[idx_v.at[pl.ds(0, nrow)]], rows0_v, gsem).wait()

        def pair(i, carry):
            ch0 = 2 * i
            g1 = pltpu.async_copy(
                table_hbm.at[idx_v.at[pl.ds((ch0 + 1) * nrow, nrow)]],
                rows1_v, gsem)
            w0 = pltpu.async_copy(
                rows0_v, feat_hbm.at[pl.ds(rbase + ch0 * nrow, nrow)], wsem)
            g1.wait()
            w0.wait()
            g2 = pltpu.async_copy(
                table_hbm.at[idx_v.at[pl.ds(((ch0 + 2) % N_CHUNK) * nrow, nrow)]],
                rows0_v, gsem)
            w1 = pltpu.async_copy(
                rows1_v, feat_hbm.at[pl.ds(rbase + (ch0 + 1) * nrow, nrow)], wsem)
            g2.wait()
            w1.wait()
            return carry

        lax.fori_loop(0, N_CHUNK // 2, pair, 0)

    return sc_kernel


_SC_GATHER = []


def _sc_gather(table, idx_flat):
    if not _SC_GATHER:
        _SC_GATHER.append(_make_sc_gather_simple())
    return _SC_GATHER[0](table, idx_flat)


P_T = 128


def _conv_body(feat_ref, x_ref, wt_ref, m_ref, st_ref):
    i = pl.program_id(0)
    feat = feat_ref[...]
    xe = jnp.broadcast_to(x_ref[...][:, None, :], (P_T, K_NN, CP))
    xe = xe.reshape(P_T * K_NN, CP)
    f6 = jnp.concatenate([feat - xe, xe], axis=1)
    y = jnp.dot(f6, wt_ref[...], preferred_element_type=jnp.float32)
    o_dim = y.shape[1]
    m_ref[...] = jnp.max(y.reshape(P_T, K_NN, o_dim), axis=1)
    acc = jnp.concatenate(
        [
            jnp.sum(y, axis=0, keepdims=True),
            jnp.sum(y * y, axis=0, keepdims=True),
            jnp.zeros((6, o_dim), jnp.float32),
        ],
        axis=0,
    )

    @pl.when(i == 0)
    def _():
        st_ref[...] = jnp.zeros_like(st_ref)

    st_ref[...] += acc


def _conv(feat, x_pad, wt_pad):
    o_dim = wt_pad.shape[1]
    return pl.pallas_call(
        _conv_body,
        grid=(BN // P_T,),
        in_specs=[
            pl.BlockSpec((P_T * K_NN, CP), lambda i: (i, 0)),
            pl.BlockSpec((P_T, CP), lambda i: (i, 0)),
            pl.BlockSpec((2 * CP, o_dim), lambda i: (0, 0)),
        ],
        out_specs=[
            pl.BlockSpec((P_T, o_dim), lambda i: (i, 0)),
            pl.BlockSpec((8, o_dim), lambda i: (0, 0)),
        ],
        out_shape=[
            jax.ShapeDtypeStruct((BN, o_dim), jnp.float32),
            jax.ShapeDtypeStruct((8, o_dim), jnp.float32),
        ],
    )(feat, x_pad, wt_pad)


R_NORM = 2048


def _norm_body(pad_w, m_ref, mu_ref, s_ref, g_ref, b_ref, out_ref):
    y = (g_ref[...] * (m_ref[...] - mu_ref[...])) / s_ref[...] + b_ref[...]
    yl = jnp.where(y >= 0, y, 0.2 * y)
    o_dim = yl.shape[1]
    if pad_w > o_dim:
        yl = jnp.concatenate(
            [yl, jnp.zeros((yl.shape[0], pad_w - o_dim), jnp.float32)], axis=1)
    out_ref[...] = yl


def _norm(m_arr, mu, s, g, b, pad_w):
    o_dim = m_arr.shape[1]
    return pl.pallas_call(
        functools.partial(_norm_body, pad_w),
        grid=(BN // R_NORM,),
        in_specs=[
            pl.BlockSpec((R_NORM, o_dim), lambda i: (i, 0)),
            pl.BlockSpec((1, o_dim), lambda i: (0, 0)),
            pl.BlockSpec((1, o_dim), lambda i: (0, 0)),
            pl.BlockSpec((1, o_dim), lambda i: (0, 0)),
            pl.BlockSpec((1, o_dim), lambda i: (0, 0)),
        ],
        out_specs=pl.BlockSpec((R_NORM, pad_w), lambda i: (i, 0)),
        out_shape=jax.ShapeDtypeStruct((BN, pad_w), jnp.float32),
    )(m_arr, mu, s, g, b)


def _gn_norm(y, gind_ref, g_ref, b_ref, cnt, leaky):
    stat = jnp.concatenate(
        [
            jnp.sum(y, axis=0, keepdims=True),
            jnp.sum(y * y, axis=0, keepdims=True),
            jnp.zeros((6, y.shape[1]), jnp.float32),
        ],
        axis=0,
    )
    gstat = jnp.dot(stat, gind_ref[...], preferred_element_type=jnp.float32)
    m = gstat[0:1, :] / cnt
    v = gstat[1:2, :] / cnt - m * m
    yn = (g_ref[...] * (y - m)) / jnp.sqrt(v + EPS) + b_ref[...]
    if leaky:
        return jnp.where(yn >= 0, yn, 0.2 * yn)
    return jnp.maximum(yn, 0.0)


def _head5_body(l_ref, w_ref, gind_ref, g_ref, b_ref,
                x5_ref, gmax_ref, gsum_ref):
    y = jnp.dot(l_ref[...], w_ref[...], preferred_element_type=jnp.float32)
    yl = _gn_norm(y, gind_ref, g_ref, b_ref, 8.0 * N, leaky=True)
    x5_ref[...] = yl
    gmax_ref[0] = jnp.max(yl, axis=0, keepdims=True)
    gsum_ref[0] = jnp.sum(yl, axis=0, keepdims=True)


def _head5(l_arr, w5_t, gind, g5, b5):
    return pl.pallas_call(
        _head5_body,
        grid=(B,),
        in_specs=[
            pl.BlockSpec((N, 512), lambda b: (b, 0)),
            pl.BlockSpec((512, 256), lambda b: (0, 0)),
            pl.BlockSpec((256, 256), lambda b: (0, 0)),
            pl.BlockSpec((1, 256), lambda b: (0, 0)),
            pl.BlockSpec((1, 256), lambda b: (0, 0)),
        ],
        out_specs=[
            pl.BlockSpec((N, 256), lambda b: (b, 0)),
            pl.BlockSpec((1, 1, 256), lambda b: (b, 0, 0)),
            pl.BlockSpec((1, 1, 256), lambda b: (b, 0, 0)),
        ],
        out_shape=[
            jax.ShapeDtypeStruct((BN, 256), jnp.float32),
            jax.ShapeDtypeStruct((B, 1, 256), jnp.float32),
            jax.ShapeDtypeStruct((B, 1, 256), jnp.float32),
        ],
    )(l_arr, w5_t, gind, g5, b5)


def _mlp1_body(l_ref, gv_ref, wa_ref, wb_ref, bm_ref, gind_ref, g_ref, b_ref,
               out_ref):
    y = jnp.dot(l_ref[...], wa_ref[...], preferred_element_type=jnp.float32)
    y = y + jnp.dot(gv_ref[0], wb_ref[...], preferred_element_type=jnp.float32)
    y = y + bm_ref[...]
    out_ref[...] = _gn_norm(y, gind_ref, g_ref, b_ref, 16.0 * N, leaky=False)


def _mlp1(l_arr, gv, wa_t, wb_t, bm, gind, g, b):
    return pl.pallas_call(
        _mlp1_body,
        grid=(B,),
        in_specs=[
            pl.BlockSpec((N, 512), lambda i: (i, 0)),
            pl.BlockSpec((1, 1, 512), lambda i: (i, 0, 0)),
            pl.BlockSpec((512, 512), lambda i: (0, 0)),
            pl.BlockSpec((512, 512), lambda i: (0, 0)),
            pl.BlockSpec((1, 512), lambda i: (0, 0)),
            pl.BlockSpec((512, 512), lambda i: (0, 0)),
            pl.BlockSpec((1, 512), lambda i: (0, 0)),
            pl.BlockSpec((1, 512), lambda i: (0, 0)),
        ],
        out_specs=pl.BlockSpec((N, 512), lambda i: (i, 0)),
        out_shape=jax.ShapeDtypeStruct((BN, 512), jnp.float32),
    )(l_arr, gv, wa_t, wb_t, bm, gind, g, b)


def _mlp_body(cnt, x_ref, w_ref, bm_ref, gind_ref, g_ref, b_ref, out_ref):
    y = jnp.dot(x_ref[...], w_ref[...], preferred_element_type=jnp.float32)
    y = y + bm_ref[...]
    out_ref[...] = _gn_norm(y, gind_ref, g_ref, b_ref, cnt, leaky=False)


def _mlp(x_arr, w_t, bm, gind, g, b, cnt):
    c_dim, o_dim = w_t.shape
    return pl.pallas_call(
        functools.partial(_mlp_body, cnt),
        grid=(B,),
        in_specs=[
            pl.BlockSpec((N, c_dim), lambda i: (i, 0)),
            pl.BlockSpec((c_dim, o_dim), lambda i: (0, 0)),
            pl.BlockSpec((1, o_dim), lambda i: (0, 0)),
            pl.BlockSpec((o_dim, o_dim), lambda i: (0, 0)),
            pl.BlockSpec((1, o_dim), lambda i: (0, 0)),
            pl.BlockSpec((1, o_dim), lambda i: (0, 0)),
        ],
        out_specs=pl.BlockSpec((N, o_dim), lambda i: (i, 0)),
        out_shape=jax.ShapeDtypeStruct((BN, o_dim), jnp.float32),
    )(x_arr, w_t, bm, gind, g, b)


def _edge_layer(x_pad, xt_pad, c_dim, w, g, b):
    o_dim = w.shape[0]
    wt_pad = jnp.zeros((2 * CP, o_dim), jnp.float32)
    wt_pad = wt_pad.at[:c_dim, :].set(jnp.transpose(w[:, :c_dim]))
    wt_pad = wt_pad.at[CP:CP + c_dim, :].set(jnp.transpose(w[:, c_dim:]))
    idx = _topk(x_pad, xt_pad)
    feat = _sc_gather(x_pad, idx.reshape(BN * K_NN))
    m_arr, stats = _conv(feat, x_pad, wt_pad)
    cnt = float(BN * K_NN)
    mu = stats[0] / cnt
    var = stats[1] / cnt - mu * mu
    s = jnp.sqrt(var + EPS)
    pad_w = CP if o_dim < CP else o_dim
    xn_pad = _norm(m_arr, mu[None, :], s[None, :], g[None, :], b[None, :], pad_w)
    xt_next = jnp.transpose(xn_pad.reshape(B, N, pad_w), (0, 2, 1))
    return xn_pad, xt_next


def _group_indicator(c_dim, groups=32):
    gsz = c_dim // groups
    ar = jnp.arange(c_dim)
    return (ar[:, None] // gsz == ar[None, :] // gsz).astype(jnp.float32)


def kernel(transpose_xyz, W1, g1, b1, W2, g2, b2, W3, g3, b3, W4, g4, b4,
           W5, g5, b5, Wm1, bm1, g6, b6, Wm2, bm2, g7, b7, Wm3, bm3, g8, b8):
    xt0 = jnp.zeros((B, CP, N), jnp.float32).at[:, :3, :].set(transpose_xyz)
    x0 = jnp.transpose(xt0, (0, 2, 1)).reshape(BN, CP)
    x1, xt1 = _edge_layer(x0, xt0, 3, W1, g1, b1)
    x2, xt2 = _edge_layer(x1, xt1, 64, W2, g2, b2)
    x3, xt3 = _edge_layer(x2, xt2, 64, W3, g3, b3)
    x4, _ = _edge_layer(x3, xt3, 128, W4, g4, b4)

    l_arr = jnp.concatenate([x1[:, :64], x2[:, :64], x3, x4], axis=1)

    x5, gmax, gsum = _head5(l_arr, jnp.transpose(W5), _group_indicator(256),
                            g5[None, :], b5[None, :])
    gv = jnp.concatenate([gmax[:, 0, :], gsum[:, 0, :] / N], axis=1)

    h = _mlp1(l_arr, gv[:, None, :], jnp.transpose(Wm1[:, :512]),
              jnp.transpose(Wm1[:, 512:]), bm1[None, :], _group_indicator(512),
              g6[None, :], b6[None, :])
    h = _mlp(h, jnp.transpose(Wm2), bm2[None, :], _group_indicator(256),
             g7[None, :], b7[None, :], 8.0 * N)
    h = _mlp(h, jnp.transpose(Wm3), bm3[None, :], _group_indicator(128),
             g8[None, :], b8[None, :], 4.0 * N)

    emb = jnp.transpose(h.reshape(B, N, 128), (0, 2, 1))
    return (emb, gv[:, :, None])
